# Initial kernel scaffold; baseline (speedup 1.0000x reference)
import jax
import jax.numpy as jnp
from jax.experimental import pallas as pl


def kernel(x_enc, Wq, bq, Wk, bk, Wv, bv, Wo, bo, c1w, c1b, c2w, c2b, g1, be1, g2, be2):
    raise NotImplementedError("write your pallas kernel here")


# speedup vs baseline: 11.2286x; 11.2286x over previous
import math

import numpy as np
import jax
import jax.numpy as jnp
from jax import lax
from jax.experimental import pallas as pl
from jax.experimental.pallas import tpu as pltpu

D_MODEL = 768
N_HEADS = 12
D_FF = 2048
E_LAYERS = 3
FACTOR = 5
B, L = 2, 2048
DH = D_MODEL // N_HEADS
U = min(int(FACTOR * math.ceil(math.log(L))), L)

NEG = -1e30


def _build_counts():
    counts = []
    for i in range(E_LAYERS):
        k = jax.random.fold_in(jax.random.key(42), i)
        idx = np.asarray(jax.random.randint(k, (L, U), 0, L))
        c = np.zeros((L, L), np.int8)
        np.add.at(c, (np.arange(L)[:, None], idx), 1)
        counts.append(c)
    return counts


_COUNTS = _build_counts()

R_QKV = 512


def _qkv_body(x_ref, w_ref, b_ref, q_ref, k_ref, v_ref):
    x = x_ref[0]
    y = lax.dot_general(x, w_ref[...], (((1,), (1,)), ((), ())),
                        preferred_element_type=jnp.float32)
    y = y + b_ref[...]
    for h in range(N_HEADS):
        q_ref[0, h] = y[:, h * DH:(h + 1) * DH]
        k_ref[0, h] = y[:, D_MODEL + h * DH:D_MODEL + (h + 1) * DH]
        v_ref[0, h] = y[:, 2 * D_MODEL + h * DH:2 * D_MODEL + (h + 1) * DH]


def _qkv(x, w_qkv, b_qkv):
    out = jax.ShapeDtypeStruct((B, N_HEADS, L, DH), jnp.float32)
    hspec = pl.BlockSpec((1, N_HEADS, R_QKV, DH), lambda b, l: (b, 0, l, 0))
    return pl.pallas_call(
        _qkv_body,
        grid=(B, L // R_QKV),
        in_specs=[
            pl.BlockSpec((1, R_QKV, D_MODEL), lambda b, l: (b, l, 0)),
            pl.BlockSpec((3 * D_MODEL, D_MODEL), lambda b, l: (0, 0)),
            pl.BlockSpec((1, 3 * D_MODEL), lambda b, l: (0, 0)),
        ],
        out_specs=[hspec, hspec, hspec],
        out_shape=[out, out, out],
    )(x, w_qkv, b_qkv)


R_M = 256


def _m_body(q_ref, k_ref, c_ref, m_ref):
    c = c_ref[...].astype(jnp.float32)
    sampled = c > 0.0
    cols = []
    for h in range(N_HEADS):
        q_h = q_ref[0, h]
        k_h = k_ref[0, h]
        s = lax.dot_general(q_h, k_h, (((1,), (1,)), ((), ())),
                            preferred_element_type=jnp.float32)
        smax = jnp.max(jnp.where(sampled, s, NEG), axis=1, keepdims=True)
        ssum = jnp.sum(s * c, axis=1, keepdims=True)
        cols.append(smax - ssum * (1.0 / L))
    m_ref[0] = jnp.concatenate(cols, axis=1)


def _measure_m(q, k, c):
    return pl.pallas_call(
        _m_body,
        grid=(B, L // R_M),
        in_specs=[
            pl.BlockSpec((1, N_HEADS, R_M, DH), lambda b, l: (b, 0, l, 0)),
            pl.BlockSpec((1, N_HEADS, L, DH), lambda b, l: (b, 0, 0, 0)),
            pl.BlockSpec((R_M, L), lambda b, l: (l, 0)),
        ],
        out_specs=pl.BlockSpec((1, R_M, N_HEADS), lambda b, l: (b, l, 0)),
        out_shape=jax.ShapeDtypeStruct((B, L, N_HEADS), jnp.float32),
    )(q, k, c)


def _topk_body(m_ref, o_ref):
    m = m_ref[...]
    iota = lax.broadcasted_iota(jnp.int32, (B * N_HEADS, L), 1)
    cols = []
    for _ in range(U):
        mx = jnp.max(m, axis=1, keepdims=True)
        eq = m >= mx
        idx = jnp.min(jnp.where(eq, iota, L), axis=1, keepdims=True)
        cols.append(idx)
        m = jnp.where(iota == idx, NEG, m)
    o_ref[...] = jnp.concatenate(cols, axis=1)


def _topk(m_bhl):
    return pl.pallas_call(
        _topk_body,
        in_specs=[pl.BlockSpec((B * N_HEADS, L), lambda: (0, 0))],
        out_specs=pl.BlockSpec((B * N_HEADS, U), lambda: (0, 0)),
        out_shape=jax.ShapeDtypeStruct((B * N_HEADS, U), jnp.int32),
    )(m_bhl)


def _attn_body(idx_ref, q_ref, k_ref, v_ref, ctx_ref, mv_ref, qr):
    for j in range(U):
        row = idx_ref[0, 0, j]
        qr[pl.ds(j, 1), :] = q_ref[0, 0, pl.ds(row, 1), :]
    k = k_ref[0, 0]
    v = v_ref[0, 0]
    s = lax.dot_general(qr[...], k, (((1,), (1,)), ((), ())),
                        preferred_element_type=jnp.float32)
    s = s * (1.0 / math.sqrt(DH))
    s = s - jnp.max(s, axis=1, keepdims=True)
    e = jnp.exp(s)
    a = e / jnp.sum(e, axis=1, keepdims=True)
    ctx_ref[0, 0] = jnp.dot(a, v, preferred_element_type=jnp.float32)
    mv_ref[0, 0] = jnp.mean(v, axis=0, keepdims=True)


def _attention(m_top, q, k, v):
    return pl.pallas_call(
        _attn_body,
        grid=(B, N_HEADS),
        in_specs=[
            pl.BlockSpec((1, 1, U), lambda b, h: (b * N_HEADS + h, 0, 0),
                         memory_space=pltpu.SMEM),
            pl.BlockSpec((1, 1, L, DH), lambda b, h: (b, h, 0, 0)),
            pl.BlockSpec((1, 1, L, DH), lambda b, h: (b, h, 0, 0)),
            pl.BlockSpec((1, 1, L, DH), lambda b, h: (b, h, 0, 0)),
        ],
        out_specs=[
            pl.BlockSpec((1, 1, U, DH), lambda b, h: (b, h, 0, 0)),
            pl.BlockSpec((1, 1, 1, DH), lambda b, h: (b, h, 0, 0)),
        ],
        out_shape=[
            jax.ShapeDtypeStruct((B, N_HEADS, U, DH), jnp.float32),
            jax.ShapeDtypeStruct((B, N_HEADS, 1, DH), jnp.float32),
        ],
        scratch_shapes=[pltpu.VMEM((U, DH), jnp.float32)],
    )(m_top, q, k, v)


def _assemble_body(idx_ref, cu_ref, mv_ref, out_ref):
    out_ref[0, 0] = jnp.broadcast_to(mv_ref[0, 0], (L, DH))
    for j in range(U):
        row = idx_ref[0, 0, j]
        out_ref[0, 0, pl.ds(row, 1), :] = cu_ref[0, 0, pl.ds(j, 1), :]


def _assemble(m_top, ctx_upd, mean_v):
    return pl.pallas_call(
        _assemble_body,
        grid=(B, N_HEADS),
        in_specs=[
            pl.BlockSpec((1, 1, U), lambda b, h: (b * N_HEADS + h, 0, 0),
                         memory_space=pltpu.SMEM),
            pl.BlockSpec((1, 1, U, DH), lambda b, h: (b, h, 0, 0)),
            pl.BlockSpec((1, 1, 1, DH), lambda b, h: (b, h, 0, 0)),
        ],
        out_specs=pl.BlockSpec((1, 1, L, DH), lambda b, h: (b, h, 0, 0)),
        out_shape=jax.ShapeDtypeStruct((B, N_HEADS, L, DH), jnp.float32),
    )(m_top, ctx_upd, mean_v)


R_T = 256


def _layer_norm(t, g, b):
    mu = jnp.mean(t, axis=1, keepdims=True)
    var = jnp.mean((t - mu) ** 2, axis=1, keepdims=True)
    return (t - mu) / jnp.sqrt(var + 1e-5) * g + b


def _tail_body(ctx_ref, x_ref, wo_ref, bo_ref, c1w_ref, c1b_ref,
               c2w_ref, c2b_ref, g1_ref, b1_ref, g2_ref, b2_ref,
               o_ref, cat):
    for h in range(N_HEADS):
        cat[:, h * DH:(h + 1) * DH] = ctx_ref[0, h]
    proj = lax.dot_general(cat[...], wo_ref[...], (((1,), (1,)), ((), ())),
                           preferred_element_type=jnp.float32)
    t = x_ref[0] + proj + bo_ref[...]
    x1 = _layer_norm(t, g1_ref[...], b1_ref[...])
    y = lax.dot_general(x1, c1w_ref[...], (((1,), (1,)), ((), ())),
                        preferred_element_type=jnp.float32)
    y = jnp.maximum(y + c1b_ref[...], 0.0)
    y = lax.dot_general(y, c2w_ref[...], (((1,), (1,)), ((), ())),
                        preferred_element_type=jnp.float32)
    y = y + c2b_ref[...]
    o_ref[0] = _layer_norm(x1 + y, g2_ref[...], b2_ref[...])


def _tail(ctx, x, wo, bo, c1w, c1b, c2w, c2b, g1, b1, g2, b2):
    full = lambda shape: pl.BlockSpec(shape, lambda b, l: tuple(0 for _ in shape))
    return pl.pallas_call(
        _tail_body,
        grid=(B, L // R_T),
        in_specs=[
            pl.BlockSpec((1, N_HEADS, R_T, DH), lambda b, l: (b, 0, l, 0)),
            pl.BlockSpec((1, R_T, D_MODEL), lambda b, l: (b, l, 0)),
            full((D_MODEL, D_MODEL)),
            full((1, D_MODEL)),
            full((D_FF, D_MODEL)),
            full((1, D_FF)),
            full((D_MODEL, D_FF)),
            full((1, D_MODEL)),
            full((1, D_MODEL)),
            full((1, D_MODEL)),
            full((1, D_MODEL)),
            full((1, D_MODEL)),
        ],
        out_specs=pl.BlockSpec((1, R_T, D_MODEL), lambda b, l: (b, l, 0)),
        out_shape=jax.ShapeDtypeStruct((B, L, D_MODEL), jnp.float32),
        scratch_shapes=[pltpu.VMEM((R_T, D_MODEL), jnp.float32)],
    )(ctx, x, wo, bo, c1w, c1b, c2w, c2b, g1, b1, g2, b2)


def kernel(x_enc, Wq, bq, Wk, bk, Wv, bv, Wo, bo, c1w, c1b, c2w, c2b,
           g1, be1, g2, be2):
    x = x_enc
    for i in range(E_LAYERS):
        w_qkv = jnp.concatenate([Wq[i], Wk[i], Wv[i]], axis=0)
        b_qkv = jnp.concatenate([bq[i], bk[i], bv[i]])[None, :]
        q, k, v = _qkv(x, w_qkv, b_qkv)
        m = _measure_m(q, k, jnp.asarray(_COUNTS[i]))
        m_bhl = m.transpose(0, 2, 1).reshape(B * N_HEADS, L)
        m_top = _topk(m_bhl).reshape(B * N_HEADS, 1, U)
        ctx_upd, mean_v = _attention(m_top, q, k, v)
        ctx = _assemble(m_top, ctx_upd, mean_v)
        x = _tail(ctx, x, Wo[i], bo[i][None, :], c1w[i], c1b[i][None, :],
                  c2w[i], c2b[i][None, :], g1[i][None, :], be1[i][None, :],
                  g2[i][None, :], be2[i][None, :])
    return x
